# baseline (speedup 1.0000x reference)
import functools

import jax
import jax.numpy as jnp
from jax import lax
from jax.experimental import pallas as pl
from jax.experimental.pallas import tpu as pltpu
from jax.experimental.pallas import tpu_sc as plsc

N = 10000
E = 320000
NG = 64
HID = 256
NC = 2
NS = 16
CH = 80


def _make_sc_aggr(edge_split):
    D = 128
    CH = 40
    if edge_split:
        EPT = E // (NS * NC)
        NB = 2
    else:
        EPT = E // NS
        NB = 4
    NCH = EPT // CH
    NCYC = NCH // NB
    RPT = 624
    REM = N - NS * RPT
    CPY = [(i * CH, CH) for i in range(RPT // CH)] + [(RPT - RPT % CH, RPT % CH)]
    mesh = plsc.VectorSubcoreMesh(
        core_axis_name="c", subcore_axis_name="s",
        num_cores=NC, num_subcores=NS)

    @functools.partial(
        pl.kernel,
        out_type=jax.ShapeDtypeStruct((NC * N, D), jnp.float32),
        mesh=mesh,
        scratch_types=[
            [pltpu.VMEM((CH,), jnp.int32) for _ in range(NB)],
            [pltpu.VMEM((CH,), jnp.int32) for _ in range(NB)],
            [pltpu.VMEM((CH, D), jnp.float32) for _ in range(NB)],
            [pltpu.VMEM((CH, D), jnp.float32) for _ in range(NB)],
            pltpu.VMEM_SHARED((N, D), jnp.float32),
            [pltpu.SemaphoreType.DMA for _ in range(NB)],
            [pltpu.SemaphoreType.DMA for _ in range(NB)],
        ],
    )
    def sc_aggr(tab_hbm, e_hbm, src_hbm, dst_hbm, out_hbm,
                srcbs, dstbs, rowss, erowss, slab, fsems, ssems):
        cid = lax.axis_index("c")
        sid = lax.axis_index("s")

        zero = jnp.zeros((16,), jnp.float32)

        def zrow(i, carry):
            for t in range(D // 16):
                rowss[0][i, pl.ds(t * 16, 16)] = zero
            return carry

        lax.fori_loop(0, CH, zrow, 0)
        for off, cnt in CPY:
            pltpu.sync_copy(rowss[0].at[pl.ds(0, cnt)],
                            slab.at[pl.ds(sid * RPT + off, cnt)])

        @pl.when(sid < 2)
        def _():
            pltpu.sync_copy(rowss[0].at[pl.ds(0, REM // 2)],
                            slab.at[pl.ds(NS * RPT + sid * (REM // 2), REM // 2)])

        plsc.subcore_barrier()

        row_off = cid * N
        tile_eb = ((sid * NC + cid) if edge_split else sid) * EPT

        def load_idx(j, b):
            eb = tile_eb + j * CH
            s_base = eb if edge_split else cid * E + eb
            pltpu.sync_copy(src_hbm.at[pl.ds(s_base, CH)], srcbs[b])
            pltpu.sync_copy(dst_hbm.at[pl.ds(eb, CH)], dstbs[b])

        def start_fetch(j, b):
            g = pltpu.async_copy(tab_hbm.at[srcbs[b]], rowss[b], fsems[b])
            e_base = (tile_eb if edge_split else cid * E + tile_eb) + j * CH
            el = pltpu.async_copy(e_hbm.at[pl.ds(e_base, CH)], erowss[b],
                                  fsems[b])
            return (g, el)

        def compute(b):
            def mrow(r, c2):
                for t in range(D // 16):
                    sl = pl.ds(t * 16, 16)
                    rowss[b][r, sl] = jnp.maximum(
                        rowss[b][r, sl] + erowss[b][r, sl], 0.0)
                return c2

            lax.fori_loop(0, CH, mrow, 0)

        def start_scat(b):
            return pltpu.async_copy(rowss[b], slab.at[dstbs[b]], ssems[b],
                                    add=True)

        def cycle(k, carry):
            fetches = []
            for b in range(NB):
                load_idx(k * NB + b, b)
                fetches.append(start_fetch(k * NB + b, b))
            scats = []
            for b in range(NB):
                fetches[b][0].wait()
                fetches[b][1].wait()
                compute(b)
                scats.append(start_scat(b))
            for b in range(NB):
                scats[b].wait()
            return carry

        lax.fori_loop(0, NCYC, cycle, 0)
        plsc.subcore_barrier()

        for off, cnt in CPY:
            rb = sid * RPT + off
            pltpu.sync_copy(slab.at[pl.ds(rb, cnt)],
                            rowss[0].at[pl.ds(0, cnt)])
            pltpu.sync_copy(rowss[0].at[pl.ds(0, cnt)],
                            out_hbm.at[pl.ds(row_off + rb, cnt)])

        @pl.when(sid < 2)
        def _():
            rb = NS * RPT + sid * (REM // 2)
            pltpu.sync_copy(slab.at[pl.ds(rb, REM // 2)],
                            rowss[0].at[pl.ds(0, REM // 2)])
            pltpu.sync_copy(rowss[0].at[pl.ds(0, REM // 2)],
                            out_hbm.at[pl.ds(row_off + rb, REM // 2)])

    return sc_aggr


_SC_CACHE = {}


def _sc_aggr(edge_split):
    if edge_split not in _SC_CACHE:
        _SC_CACHE[edge_split] = _make_sc_aggr(edge_split)
    return _SC_CACHE[edge_split]


def _edge_lin(ea, W, b, nh):
    BE = 8000
    GE = E // BE
    half = 128

    def body(ea_ref, w_ref, b_ref, out_ref):
        out_ref[...] = (
            jnp.dot(ea_ref[...], w_ref[...], preferred_element_type=jnp.float32)
            + b_ref[...])

    Wh = W.reshape(16, nh, half).transpose(1, 0, 2)
    bh = b.reshape(1, nh, half).transpose(1, 0, 2)
    return pl.pallas_call(
        body,
        grid=(nh, GE),
        in_specs=[
            pl.BlockSpec((BE, 16), lambda h, i: (i, 0)),
            pl.BlockSpec((None, 16, half), lambda h, i: (h, 0, 0)),
            pl.BlockSpec((None, 1, half), lambda h, i: (h, 0, 0)),
        ],
        out_specs=pl.BlockSpec((BE, half), lambda h, i: (h * GE + i, 0)),
        out_shape=jax.ShapeDtypeStruct((nh * E, half), jnp.float32),
    )(ea, Wh, bh)


def _dense_a(h_arr, aggr_split, W1, b1, W2, b2, hin, layer0):
    half = 128
    NR = 1000
    G = N // NR

    def body(hlo, hhi, alo, ahi, w1, bb1, w2, bb2, v_ref, s1_ref, s2_ref):
        i = pl.program_id(0)
        if layer0:
            u = hlo[...] + alo[...] + ahi[...]
        else:
            u = jnp.concatenate(
                [hlo[...] + alo[...], hhi[...] + ahi[...]], axis=1)
        t = jnp.maximum(
            jnp.dot(u, w1[...], preferred_element_type=jnp.float32) + bb1[...],
            0.0)
        v = jnp.dot(t, w2[...], preferred_element_type=jnp.float32) + bb2[...]
        v_ref[...] = v
        s1 = jnp.sum(v, axis=0, keepdims=True)
        s2 = jnp.sum(v * v, axis=0, keepdims=True)

        @pl.when(i == 0)
        def _():
            s1_ref[...] = s1
            s2_ref[...] = s2

        @pl.when(i > 0)
        def _():
            s1_ref[...] += s1
            s2_ref[...] += s2

    return pl.pallas_call(
        body,
        grid=(G,),
        in_specs=[
            pl.BlockSpec((NR, half), lambda i: (i, 0)),
            pl.BlockSpec((NR, half),
                         (lambda i: (i, 0)) if layer0 else (lambda i: (G + i, 0))),
            pl.BlockSpec((NR, half), lambda i: (i, 0)),
            pl.BlockSpec((NR, half), lambda i: (G + i, 0)),
            pl.BlockSpec((hin, HID), lambda i: (0, 0)),
            pl.BlockSpec((1, HID), lambda i: (0, 0)),
            pl.BlockSpec((HID, HID), lambda i: (0, 0)),
            pl.BlockSpec((1, HID), lambda i: (0, 0)),
        ],
        out_specs=[
            pl.BlockSpec((NR, HID), lambda i: (i, 0)),
            pl.BlockSpec((1, HID), lambda i: (0, 0)),
            pl.BlockSpec((1, HID), lambda i: (0, 0)),
        ],
        out_shape=[
            jax.ShapeDtypeStruct((N, HID), jnp.float32),
            jax.ShapeDtypeStruct((1, HID), jnp.float32),
            jax.ShapeDtypeStruct((1, HID), jnp.float32),
        ],
    )(h_arr, h_arr, aggr_split, aggr_split, W1, b1.reshape(1, -1),
      W2, b2.reshape(1, -1))


def _dense_b(v, s1, s2, bn_g, bn_b):
    NR = 1000
    G = N // NR

    def body(v_ref, s1_ref, s2_ref, g_ref, b_ref, out_ref):
        mean = s1_ref[...] * (1.0 / N)
        var = s2_ref[...] * (1.0 / N) - mean * mean
        inv = lax.rsqrt(var + 1e-5)
        out_ref[...] = jnp.maximum(
            (v_ref[...] - mean) * inv * g_ref[...] + b_ref[...], 0.0)

    return pl.pallas_call(
        body,
        grid=(2, G),
        in_specs=[
            pl.BlockSpec((NR, HID // 2), lambda h, i: (i, h)),
            pl.BlockSpec((1, HID // 2), lambda h, i: (0, h)),
            pl.BlockSpec((1, HID // 2), lambda h, i: (0, h)),
            pl.BlockSpec((1, HID // 2), lambda h, i: (0, h)),
            pl.BlockSpec((1, HID // 2), lambda h, i: (0, h)),
        ],
        out_specs=pl.BlockSpec((NR, HID // 2), lambda h, i: (h * G + i, 0)),
        out_shape=jax.ShapeDtypeStruct((2 * N, HID // 2), jnp.float32),
    )(v, s1, s2, bn_g.reshape(1, -1), bn_b.reshape(1, -1))


def _softplus(x):
    return jnp.maximum(x, 0.0) + jnp.log1p(jnp.exp(-jnp.abs(x)))


def _head(h_split, batch2, global_feat, p):
    def body(h_ref, b_ref, gf_ref, wp1, bp1, wp2, bp2, wf1, bf1, wf2, bf2,
             we, be, edl_ref, z_ref, g_ref):
        h = jnp.concatenate([h_ref[:N, :], h_ref[N:, :]], axis=1)
        seg = lax.broadcasted_iota(jnp.int32, (N, NG), 1)
        onehot = (b_ref[...] == seg).astype(jnp.float32)
        g = lax.dot_general(onehot, h, (((0,), (0,)), ((), ())),
                            preferred_element_type=jnp.float32)
        g_ref[...] = g
        z = jnp.dot(
            jnp.maximum(jnp.dot(g, wp1[...],
                                preferred_element_type=jnp.float32) + bp1[...],
                        0.0),
            wp2[...], preferred_element_type=jnp.float32) + bp2[...]
        nrm = jnp.sqrt(jnp.sum(z * z, axis=1, keepdims=True))
        z_ref[...] = z / jnp.maximum(nrm, 1e-12)
        gc = jnp.concatenate([g, gf_ref[...]], axis=1)
        gf1 = jnp.maximum(
            jnp.dot(gc, wf1[...], preferred_element_type=jnp.float32)
            + bf1[...], 0.0)
        gf2 = jnp.maximum(
            jnp.dot(gf1, wf2[...], preferred_element_type=jnp.float32)
            + bf2[...], 0.0)
        out = jnp.dot(gf2, we[...], preferred_element_type=jnp.float32) + be[...]
        gamma = out[:, 0:1]
        nu = _softplus(out[:, 1:2]) + 1e-06
        alpha = _softplus(out[:, 2:3]) + 1.0 + 1e-06
        beta = _softplus(out[:, 3:4]) + 1e-06
        edl_ref[...] = jnp.concatenate([gamma, nu, alpha, beta], axis=1)

    return pl.pallas_call(
        body,
        out_shape=[
            jax.ShapeDtypeStruct((NG, 4), jnp.float32),
            jax.ShapeDtypeStruct((NG, NG), jnp.float32),
            jax.ShapeDtypeStruct((NG, HID), jnp.float32),
        ],
    )(h_split, batch2, global_feat, p['Wp1'], p['bp1'].reshape(1, -1),
      p['Wp2'], p['bp2'].reshape(1, -1), p['Wf1'], p['bf1'].reshape(1, -1),
      p['Wf2'], p['bf2'].reshape(1, -1), p['We'], p['be'].reshape(1, -1))


def kernel(x, edge_index, edge_attr, batch, global_feat, params):
    src = edge_index[0].astype(jnp.int32)
    dst = edge_index[1].astype(jnp.int32)
    batch2 = batch.astype(jnp.int32).reshape(N, 1)

    src2 = jnp.concatenate([src, src + N])

    h_split = x
    for l in range(4):
        layer0 = l == 0
        hin = 128 if layer0 else HID
        elin = _edge_lin(edge_attr, params['edge_W_%d' % l],
                         params['edge_b_%d' % l], 1 if layer0 else 2)
        aggr_split = _sc_aggr(layer0)(h_split, elin,
                                      src if layer0 else src2, dst)
        v, s1, s2 = _dense_a(h_split, aggr_split, params['W1_%d' % l],
                             params['b1_%d' % l], params['W2_%d' % l],
                             params['b2_%d' % l], hin, layer0)
        h_split = _dense_b(v, s1, s2, params['bn_g_%d' % l],
                           params['bn_b_%d' % l])

    edl, z, g = _head(h_split, batch2, global_feat, params)
    return (edl, z, g)


# speedup vs baseline: 1.0729x; 1.0729x over previous
import functools

import jax
import jax.numpy as jnp
from jax import lax
from jax.experimental import pallas as pl
from jax.experimental.pallas import tpu as pltpu
from jax.experimental.pallas import tpu_sc as plsc

N = 10000
E = 320000
NG = 64
HID = 256
NC = 2
NS = 16
CH = 80


def _make_sc_aggr(edge_split):
    D = 128
    if edge_split:
        CH = 40
        EPT = E // (NS * NC)
        NB = 2
    else:
        CH = 32
        EPT = E // NS
        NB = 5
    NCH = EPT // CH
    NCYC = NCH // NB
    RPT = 624
    REM = N - NS * RPT
    CPY = [(i * CH, CH) for i in range(RPT // CH)] + [(RPT - RPT % CH, RPT % CH)]
    mesh = plsc.VectorSubcoreMesh(
        core_axis_name="c", subcore_axis_name="s",
        num_cores=NC, num_subcores=NS)

    @functools.partial(
        pl.kernel,
        out_type=jax.ShapeDtypeStruct((NC * N, D), jnp.float32),
        mesh=mesh,
        scratch_types=[
            pltpu.VMEM((NB * CH,), jnp.int32),
            pltpu.VMEM((NB * CH,), jnp.int32),
            [pltpu.VMEM((CH,), jnp.int32) for _ in range(NB)],
            [pltpu.VMEM((CH, D), jnp.float32) for _ in range(NB)],
            [pltpu.VMEM((CH, D), jnp.float32) for _ in range(NB)],
            pltpu.VMEM_SHARED((N, D), jnp.float32),
            [pltpu.SemaphoreType.DMA for _ in range(NB)],
            [pltpu.SemaphoreType.DMA for _ in range(NB)],
        ],
    )
    def sc_aggr(tab_hbm, e_hbm, src_hbm, dst_hbm, out_hbm,
                srcball, dstball, dstbs, rowss, erowss, slab, fsems, ssems):
        cid = lax.axis_index("c")
        sid = lax.axis_index("s")

        zero = jnp.zeros((16,), jnp.float32)

        def zrow(i, carry):
            for t in range(D // 16):
                rowss[0][i, pl.ds(t * 16, 16)] = zero
            return carry

        lax.fori_loop(0, CH, zrow, 0)
        for off, cnt in CPY:
            pltpu.sync_copy(rowss[0].at[pl.ds(0, cnt)],
                            slab.at[pl.ds(sid * RPT + off, cnt)])

        @pl.when(sid < 2)
        def _():
            pltpu.sync_copy(rowss[0].at[pl.ds(0, REM // 2)],
                            slab.at[pl.ds(NS * RPT + sid * (REM // 2), REM // 2)])

        plsc.subcore_barrier()

        row_off = cid * N
        tile_eb = ((sid * NC + cid) if edge_split else sid) * EPT

        def load_idx_cycle(k):
            eb = tile_eb + k * (NB * CH)
            s_base = eb if edge_split else cid * E + eb
            pltpu.sync_copy(src_hbm.at[pl.ds(s_base, NB * CH)], srcball)
            if edge_split:
                for b in range(NB):
                    pltpu.sync_copy(dst_hbm.at[pl.ds(eb + b * CH, CH)],
                                    dstbs[b])
            else:
                pltpu.sync_copy(dst_hbm.at[pl.ds(eb, NB * CH)], dstball)
                for b in range(NB):
                    for t in range(CH // 16):
                        dstbs[b][pl.ds(t * 16, 16)] = (
                            dstball[pl.ds(b * CH + t * 16, 16)])

        def start_fetch(k, b):
            g = pltpu.async_copy(tab_hbm.at[srcball.at[pl.ds(b * CH, CH)]],
                                 rowss[b], fsems[b])
            e_base = ((tile_eb if edge_split else cid * E + tile_eb)
                      + (k * NB + b) * CH)
            el = pltpu.async_copy(e_hbm.at[pl.ds(e_base, CH)], erowss[b],
                                  fsems[b])
            return (g, el)

        def compute(b):
            def mrow(r, c2):
                for t in range(D // 16):
                    sl = pl.ds(t * 16, 16)
                    rowss[b][r, sl] = jnp.maximum(
                        rowss[b][r, sl] + erowss[b][r, sl], 0.0)
                return c2

            lax.fori_loop(0, CH, mrow, 0)

        def start_scat(b):
            return pltpu.async_copy(rowss[b], slab.at[dstbs[b]], ssems[b],
                                    add=True)

        def cycle(k, carry):
            load_idx_cycle(k)
            fetches = []
            for b in range(NB):
                fetches.append(start_fetch(k, b))
            scats = []
            for b in range(NB):
                fetches[b][0].wait()
                fetches[b][1].wait()
                compute(b)
                scats.append(start_scat(b))
            for b in range(NB):
                scats[b].wait()
            return carry

        lax.fori_loop(0, NCYC, cycle, 0)
        plsc.subcore_barrier()

        for off, cnt in CPY:
            rb = sid * RPT + off
            pltpu.sync_copy(slab.at[pl.ds(rb, cnt)],
                            rowss[0].at[pl.ds(0, cnt)])
            pltpu.sync_copy(rowss[0].at[pl.ds(0, cnt)],
                            out_hbm.at[pl.ds(row_off + rb, cnt)])

        @pl.when(sid < 2)
        def _():
            rb = NS * RPT + sid * (REM // 2)
            pltpu.sync_copy(slab.at[pl.ds(rb, REM // 2)],
                            rowss[0].at[pl.ds(0, REM // 2)])
            pltpu.sync_copy(rowss[0].at[pl.ds(0, REM // 2)],
                            out_hbm.at[pl.ds(row_off + rb, REM // 2)])

    return sc_aggr


_SC_CACHE = {}


def _sc_aggr(edge_split):
    if edge_split not in _SC_CACHE:
        _SC_CACHE[edge_split] = _make_sc_aggr(edge_split)
    return _SC_CACHE[edge_split]


def _edge_lin(ea, W, b, nh):
    BE = 8000
    GE = E // BE
    half = 128

    def body(ea_ref, w_ref, b_ref, out_ref):
        out_ref[...] = (
            jnp.dot(ea_ref[...], w_ref[...], preferred_element_type=jnp.float32)
            + b_ref[...])

    Wh = W.reshape(16, nh, half).transpose(1, 0, 2)
    bh = b.reshape(1, nh, half).transpose(1, 0, 2)
    return pl.pallas_call(
        body,
        grid=(nh, GE),
        in_specs=[
            pl.BlockSpec((BE, 16), lambda h, i: (i, 0)),
            pl.BlockSpec((None, 16, half), lambda h, i: (h, 0, 0)),
            pl.BlockSpec((None, 1, half), lambda h, i: (h, 0, 0)),
        ],
        out_specs=pl.BlockSpec((BE, half), lambda h, i: (h * GE + i, 0)),
        out_shape=jax.ShapeDtypeStruct((nh * E, half), jnp.float32),
    )(ea, Wh, bh)


def _dense_a(h_arr, aggr_split, W1, b1, W2, b2, hin, layer0):
    half = 128
    NR = 1000
    G = N // NR

    def body(hlo, hhi, alo, ahi, w1, bb1, w2, bb2, v_ref, s1_ref, s2_ref):
        i = pl.program_id(0)
        if layer0:
            u = hlo[...] + alo[...] + ahi[...]
        else:
            u = jnp.concatenate(
                [hlo[...] + alo[...], hhi[...] + ahi[...]], axis=1)
        t = jnp.maximum(
            jnp.dot(u, w1[...], preferred_element_type=jnp.float32) + bb1[...],
            0.0)
        v = jnp.dot(t, w2[...], preferred_element_type=jnp.float32) + bb2[...]
        v_ref[...] = v
        s1 = jnp.sum(v, axis=0, keepdims=True)
        s2 = jnp.sum(v * v, axis=0, keepdims=True)

        @pl.when(i == 0)
        def _():
            s1_ref[...] = s1
            s2_ref[...] = s2

        @pl.when(i > 0)
        def _():
            s1_ref[...] += s1
            s2_ref[...] += s2

    return pl.pallas_call(
        body,
        grid=(G,),
        in_specs=[
            pl.BlockSpec((NR, half), lambda i: (i, 0)),
            pl.BlockSpec((NR, half),
                         (lambda i: (i, 0)) if layer0 else (lambda i: (G + i, 0))),
            pl.BlockSpec((NR, half), lambda i: (i, 0)),
            pl.BlockSpec((NR, half), lambda i: (G + i, 0)),
            pl.BlockSpec((hin, HID), lambda i: (0, 0)),
            pl.BlockSpec((1, HID), lambda i: (0, 0)),
            pl.BlockSpec((HID, HID), lambda i: (0, 0)),
            pl.BlockSpec((1, HID), lambda i: (0, 0)),
        ],
        out_specs=[
            pl.BlockSpec((NR, HID), lambda i: (i, 0)),
            pl.BlockSpec((1, HID), lambda i: (0, 0)),
            pl.BlockSpec((1, HID), lambda i: (0, 0)),
        ],
        out_shape=[
            jax.ShapeDtypeStruct((N, HID), jnp.float32),
            jax.ShapeDtypeStruct((1, HID), jnp.float32),
            jax.ShapeDtypeStruct((1, HID), jnp.float32),
        ],
    )(h_arr, h_arr, aggr_split, aggr_split, W1, b1.reshape(1, -1),
      W2, b2.reshape(1, -1))


def _dense_b(v, s1, s2, bn_g, bn_b):
    NR = 1000
    G = N // NR

    def body(v_ref, s1_ref, s2_ref, g_ref, b_ref, out_ref):
        mean = s1_ref[...] * (1.0 / N)
        var = s2_ref[...] * (1.0 / N) - mean * mean
        inv = lax.rsqrt(var + 1e-5)
        out_ref[...] = jnp.maximum(
            (v_ref[...] - mean) * inv * g_ref[...] + b_ref[...], 0.0)

    return pl.pallas_call(
        body,
        grid=(2, G),
        in_specs=[
            pl.BlockSpec((NR, HID // 2), lambda h, i: (i, h)),
            pl.BlockSpec((1, HID // 2), lambda h, i: (0, h)),
            pl.BlockSpec((1, HID // 2), lambda h, i: (0, h)),
            pl.BlockSpec((1, HID // 2), lambda h, i: (0, h)),
            pl.BlockSpec((1, HID // 2), lambda h, i: (0, h)),
        ],
        out_specs=pl.BlockSpec((NR, HID // 2), lambda h, i: (h * G + i, 0)),
        out_shape=jax.ShapeDtypeStruct((2 * N, HID // 2), jnp.float32),
    )(v, s1, s2, bn_g.reshape(1, -1), bn_b.reshape(1, -1))


def _softplus(x):
    return jnp.maximum(x, 0.0) + jnp.log1p(jnp.exp(-jnp.abs(x)))


def _head(h_split, batch2, global_feat, p):
    def body(h_ref, b_ref, gf_ref, wp1, bp1, wp2, bp2, wf1, bf1, wf2, bf2,
             we, be, edl_ref, z_ref, g_ref):
        h = jnp.concatenate([h_ref[:N, :], h_ref[N:, :]], axis=1)
        seg = lax.broadcasted_iota(jnp.int32, (N, NG), 1)
        onehot = (b_ref[...] == seg).astype(jnp.float32)
        g = lax.dot_general(onehot, h, (((0,), (0,)), ((), ())),
                            preferred_element_type=jnp.float32)
        g_ref[...] = g
        z = jnp.dot(
            jnp.maximum(jnp.dot(g, wp1[...],
                                preferred_element_type=jnp.float32) + bp1[...],
                        0.0),
            wp2[...], preferred_element_type=jnp.float32) + bp2[...]
        nrm = jnp.sqrt(jnp.sum(z * z, axis=1, keepdims=True))
        z_ref[...] = z / jnp.maximum(nrm, 1e-12)
        gc = jnp.concatenate([g, gf_ref[...]], axis=1)
        gf1 = jnp.maximum(
            jnp.dot(gc, wf1[...], preferred_element_type=jnp.float32)
            + bf1[...], 0.0)
        gf2 = jnp.maximum(
            jnp.dot(gf1, wf2[...], preferred_element_type=jnp.float32)
            + bf2[...], 0.0)
        out = jnp.dot(gf2, we[...], preferred_element_type=jnp.float32) + be[...]
        gamma = out[:, 0:1]
        nu = _softplus(out[:, 1:2]) + 1e-06
        alpha = _softplus(out[:, 2:3]) + 1.0 + 1e-06
        beta = _softplus(out[:, 3:4]) + 1e-06
        edl_ref[...] = jnp.concatenate([gamma, nu, alpha, beta], axis=1)

    return pl.pallas_call(
        body,
        out_shape=[
            jax.ShapeDtypeStruct((NG, 4), jnp.float32),
            jax.ShapeDtypeStruct((NG, NG), jnp.float32),
            jax.ShapeDtypeStruct((NG, HID), jnp.float32),
        ],
    )(h_split, batch2, global_feat, p['Wp1'], p['bp1'].reshape(1, -1),
      p['Wp2'], p['bp2'].reshape(1, -1), p['Wf1'], p['bf1'].reshape(1, -1),
      p['Wf2'], p['bf2'].reshape(1, -1), p['We'], p['be'].reshape(1, -1))


def kernel(x, edge_index, edge_attr, batch, global_feat, params):
    src = edge_index[0].astype(jnp.int32)
    dst = edge_index[1].astype(jnp.int32)
    batch2 = batch.astype(jnp.int32).reshape(N, 1)

    src2 = jnp.concatenate([src, src + N])

    h_split = x
    for l in range(4):
        layer0 = l == 0
        hin = 128 if layer0 else HID
        elin = _edge_lin(edge_attr, params['edge_W_%d' % l],
                         params['edge_b_%d' % l], 1 if layer0 else 2)
        aggr_split = _sc_aggr(layer0)(h_split, elin,
                                      src if layer0 else src2, dst)
        v, s1, s2 = _dense_a(h_split, aggr_split, params['W1_%d' % l],
                             params['b1_%d' % l], params['W2_%d' % l],
                             params['b2_%d' % l], hin, layer0)
        h_split = _dense_b(v, s1, s2, params['bn_g_%d' % l],
                           params['bn_b_%d' % l])

    edl, z, g = _head(h_split, batch2, global_feat, params)
    return (edl, z, g)


# speedup vs baseline: 1.2015x; 1.1199x over previous
import functools

import jax
import jax.numpy as jnp
from jax import lax
from jax.experimental import pallas as pl
from jax.experimental.pallas import tpu as pltpu
from jax.experimental.pallas import tpu_sc as plsc

N = 10000
E = 320000
NG = 64
HID = 256
NC = 2
NS = 16
CH = 80


def _make_sc_aggr(edge_split):
    D = 128
    if edge_split:
        CH = 40
        EPT = E // (NS * NC)
        NB = 2
    else:
        CH = 32
        EPT = E // NS
        NB = 5
    NCH = EPT // CH
    NCYC = NCH // NB
    RPT = 624
    REM = N - NS * RPT
    CPY = [(i * CH, CH) for i in range(RPT // CH)] + [(RPT - RPT % CH, RPT % CH)]
    mesh = plsc.VectorSubcoreMesh(
        core_axis_name="c", subcore_axis_name="s",
        num_cores=NC, num_subcores=NS)

    @functools.partial(
        pl.kernel,
        out_type=jax.ShapeDtypeStruct((NC * N, D), jnp.float32),
        mesh=mesh,
        scratch_types=[
            pltpu.VMEM((NB * CH,), jnp.int32),
            pltpu.VMEM((NB * CH,), jnp.int32),
            [pltpu.VMEM((CH,), jnp.int32) for _ in range(NB)],
            [pltpu.VMEM((CH, D), jnp.float32) for _ in range(NB)],
            [pltpu.VMEM((CH, D), jnp.float32) for _ in range(NB)],
            pltpu.VMEM_SHARED((N, D), jnp.float32),
            [pltpu.SemaphoreType.DMA for _ in range(NB)],
            [pltpu.SemaphoreType.DMA for _ in range(NB)],
        ],
    )
    def sc_aggr(tab_hbm, e_hbm, src_hbm, dst_hbm, out_hbm,
                srcball, dstball, dstbs, rowss, erowss, slab, fsems, ssems):
        cid = lax.axis_index("c")
        sid = lax.axis_index("s")

        zero = jnp.zeros((16,), jnp.float32)

        def zrow(i, carry):
            for t in range(D // 16):
                rowss[0][i, pl.ds(t * 16, 16)] = zero
            return carry

        lax.fori_loop(0, CH, zrow, 0)
        for off, cnt in CPY:
            pltpu.sync_copy(rowss[0].at[pl.ds(0, cnt)],
                            slab.at[pl.ds(sid * RPT + off, cnt)])

        @pl.when(sid < 2)
        def _():
            pltpu.sync_copy(rowss[0].at[pl.ds(0, REM // 2)],
                            slab.at[pl.ds(NS * RPT + sid * (REM // 2), REM // 2)])

        plsc.subcore_barrier()

        row_off = cid * N
        tile_eb = ((sid * NC + cid) if edge_split else sid) * EPT

        def load_idx_cycle(k):
            eb = tile_eb + k * (NB * CH)
            s_base = eb if edge_split else cid * E + eb
            pltpu.sync_copy(src_hbm.at[pl.ds(s_base, NB * CH)], srcball)

        def start_fetch(k, b):
            g = pltpu.async_copy(tab_hbm.at[srcball.at[pl.ds(b * CH, CH)]],
                                 rowss[b], fsems[b])
            e_base = ((tile_eb if edge_split else cid * E + tile_eb)
                      + (k * NB + b) * CH)
            el = pltpu.async_copy(e_hbm.at[pl.ds(e_base, CH)], erowss[b],
                                  fsems[b])
            return (g, el)

        def compute(b):
            def mrow(r, c2):
                for t in range(D // 16):
                    sl = pl.ds(t * 16, 16)
                    rowss[b][r, sl] = jnp.maximum(
                        rowss[b][r, sl] + erowss[b][r, sl], 0.0)
                return c2

            lax.fori_loop(0, CH, mrow, 0)

        def start_scat(b):
            return pltpu.async_copy(rowss[b], slab.at[dstbs[b]], ssems[b],
                                    add=True)

        def load_dst_cycle(k):
            eb = tile_eb + k * (NB * CH)
            if edge_split:
                for b in range(NB):
                    pltpu.sync_copy(dst_hbm.at[pl.ds(eb + b * CH, CH)],
                                    dstbs[b])
            else:
                pltpu.sync_copy(dst_hbm.at[pl.ds(eb, NB * CH)], dstball)
                for b in range(NB):
                    for t in range(CH // 16):
                        dstbs[b][pl.ds(t * 16, 16)] = (
                            dstball[pl.ds(b * CH + t * 16, 16)])

        def cycle(k, carry):
            load_idx_cycle(k)
            fetches = []
            for b in range(NB):
                fetches.append(start_fetch(k, b))
            load_dst_cycle(k)
            scats = []
            for b in range(NB):
                fetches[b][0].wait()
                fetches[b][1].wait()
                compute(b)
                scats.append(start_scat(b))
            for b in range(NB):
                scats[b].wait()
            return carry

        lax.fori_loop(0, NCYC, cycle, 0)
        plsc.subcore_barrier()

        for off, cnt in CPY:
            rb = sid * RPT + off
            pltpu.sync_copy(slab.at[pl.ds(rb, cnt)],
                            rowss[0].at[pl.ds(0, cnt)])
            pltpu.sync_copy(rowss[0].at[pl.ds(0, cnt)],
                            out_hbm.at[pl.ds(row_off + rb, cnt)])

        @pl.when(sid < 2)
        def _():
            rb = NS * RPT + sid * (REM // 2)
            pltpu.sync_copy(slab.at[pl.ds(rb, REM // 2)],
                            rowss[0].at[pl.ds(0, REM // 2)])
            pltpu.sync_copy(rowss[0].at[pl.ds(0, REM // 2)],
                            out_hbm.at[pl.ds(row_off + rb, REM // 2)])

    return sc_aggr


_SC_CACHE = {}


def _sc_aggr(edge_split):
    if edge_split not in _SC_CACHE:
        _SC_CACHE[edge_split] = _make_sc_aggr(edge_split)
    return _SC_CACHE[edge_split]


def _edge_lin(ea, W, b, nh):
    BE = 8000
    GE = E // BE
    half = 128

    def body(ea_ref, w_ref, b_ref, out_ref):
        out_ref[...] = (
            jnp.dot(ea_ref[...], w_ref[...], preferred_element_type=jnp.float32)
            + b_ref[...])

    Wh = W.reshape(16, nh, half).transpose(1, 0, 2)
    bh = b.reshape(1, nh, half).transpose(1, 0, 2)
    return pl.pallas_call(
        body,
        grid=(nh, GE),
        in_specs=[
            pl.BlockSpec((BE, 16), lambda h, i: (i, 0)),
            pl.BlockSpec((None, 16, half), lambda h, i: (h, 0, 0)),
            pl.BlockSpec((None, 1, half), lambda h, i: (h, 0, 0)),
        ],
        out_specs=pl.BlockSpec((BE, half), lambda h, i: (h * GE + i, 0)),
        out_shape=jax.ShapeDtypeStruct((nh * E, half), jnp.float32),
    )(ea, Wh, bh)


def _dense_a(h_arr, aggr_split, W1, b1, W2, b2, hin, layer0):
    half = 128
    NR = 1000
    G = N // NR

    def body(hlo, hhi, alo, ahi, w1, bb1, w2, bb2, v_ref, s1_ref, s2_ref):
        i = pl.program_id(0)
        if layer0:
            u = hlo[...] + alo[...] + ahi[...]
        else:
            u = jnp.concatenate(
                [hlo[...] + alo[...], hhi[...] + ahi[...]], axis=1)
        t = jnp.maximum(
            jnp.dot(u, w1[...], preferred_element_type=jnp.float32) + bb1[...],
            0.0)
        v = jnp.dot(t, w2[...], preferred_element_type=jnp.float32) + bb2[...]
        v_ref[...] = v
        s1 = jnp.sum(v, axis=0, keepdims=True)
        s2 = jnp.sum(v * v, axis=0, keepdims=True)

        @pl.when(i == 0)
        def _():
            s1_ref[...] = s1
            s2_ref[...] = s2

        @pl.when(i > 0)
        def _():
            s1_ref[...] += s1
            s2_ref[...] += s2

    return pl.pallas_call(
        body,
        grid=(G,),
        in_specs=[
            pl.BlockSpec((NR, half), lambda i: (i, 0)),
            pl.BlockSpec((NR, half),
                         (lambda i: (i, 0)) if layer0 else (lambda i: (G + i, 0))),
            pl.BlockSpec((NR, half), lambda i: (i, 0)),
            pl.BlockSpec((NR, half), lambda i: (G + i, 0)),
            pl.BlockSpec((hin, HID), lambda i: (0, 0)),
            pl.BlockSpec((1, HID), lambda i: (0, 0)),
            pl.BlockSpec((HID, HID), lambda i: (0, 0)),
            pl.BlockSpec((1, HID), lambda i: (0, 0)),
        ],
        out_specs=[
            pl.BlockSpec((NR, HID), lambda i: (i, 0)),
            pl.BlockSpec((1, HID), lambda i: (0, 0)),
            pl.BlockSpec((1, HID), lambda i: (0, 0)),
        ],
        out_shape=[
            jax.ShapeDtypeStruct((N, HID), jnp.float32),
            jax.ShapeDtypeStruct((1, HID), jnp.float32),
            jax.ShapeDtypeStruct((1, HID), jnp.float32),
        ],
    )(h_arr, h_arr, aggr_split, aggr_split, W1, b1.reshape(1, -1),
      W2, b2.reshape(1, -1))


def _dense_b(v, s1, s2, bn_g, bn_b):
    NR = 1000
    G = N // NR

    def body(v_ref, s1_ref, s2_ref, g_ref, b_ref, out_ref):
        mean = s1_ref[...] * (1.0 / N)
        var = s2_ref[...] * (1.0 / N) - mean * mean
        inv = lax.rsqrt(var + 1e-5)
        out_ref[...] = jnp.maximum(
            (v_ref[...] - mean) * inv * g_ref[...] + b_ref[...], 0.0)

    return pl.pallas_call(
        body,
        grid=(2, G),
        in_specs=[
            pl.BlockSpec((NR, HID // 2), lambda h, i: (i, h)),
            pl.BlockSpec((1, HID // 2), lambda h, i: (0, h)),
            pl.BlockSpec((1, HID // 2), lambda h, i: (0, h)),
            pl.BlockSpec((1, HID // 2), lambda h, i: (0, h)),
            pl.BlockSpec((1, HID // 2), lambda h, i: (0, h)),
        ],
        out_specs=pl.BlockSpec((NR, HID // 2), lambda h, i: (h * G + i, 0)),
        out_shape=jax.ShapeDtypeStruct((2 * N, HID // 2), jnp.float32),
    )(v, s1, s2, bn_g.reshape(1, -1), bn_b.reshape(1, -1))


def _softplus(x):
    return jnp.maximum(x, 0.0) + jnp.log1p(jnp.exp(-jnp.abs(x)))


def _head(h_split, batch2, global_feat, p):
    def body(h_ref, b_ref, gf_ref, wp1, bp1, wp2, bp2, wf1, bf1, wf2, bf2,
             we, be, edl_ref, z_ref, g_ref):
        h = jnp.concatenate([h_ref[:N, :], h_ref[N:, :]], axis=1)
        seg = lax.broadcasted_iota(jnp.int32, (N, NG), 1)
        onehot = (b_ref[...] == seg).astype(jnp.float32)
        g = lax.dot_general(onehot, h, (((0,), (0,)), ((), ())),
                            preferred_element_type=jnp.float32)
        g_ref[...] = g
        z = jnp.dot(
            jnp.maximum(jnp.dot(g, wp1[...],
                                preferred_element_type=jnp.float32) + bp1[...],
                        0.0),
            wp2[...], preferred_element_type=jnp.float32) + bp2[...]
        nrm = jnp.sqrt(jnp.sum(z * z, axis=1, keepdims=True))
        z_ref[...] = z / jnp.maximum(nrm, 1e-12)
        gc = jnp.concatenate([g, gf_ref[...]], axis=1)
        gf1 = jnp.maximum(
            jnp.dot(gc, wf1[...], preferred_element_type=jnp.float32)
            + bf1[...], 0.0)
        gf2 = jnp.maximum(
            jnp.dot(gf1, wf2[...], preferred_element_type=jnp.float32)
            + bf2[...], 0.0)
        out = jnp.dot(gf2, we[...], preferred_element_type=jnp.float32) + be[...]
        gamma = out[:, 0:1]
        nu = _softplus(out[:, 1:2]) + 1e-06
        alpha = _softplus(out[:, 2:3]) + 1.0 + 1e-06
        beta = _softplus(out[:, 3:4]) + 1e-06
        edl_ref[...] = jnp.concatenate([gamma, nu, alpha, beta], axis=1)

    return pl.pallas_call(
        body,
        out_shape=[
            jax.ShapeDtypeStruct((NG, 4), jnp.float32),
            jax.ShapeDtypeStruct((NG, NG), jnp.float32),
            jax.ShapeDtypeStruct((NG, HID), jnp.float32),
        ],
    )(h_split, batch2, global_feat, p['Wp1'], p['bp1'].reshape(1, -1),
      p['Wp2'], p['bp2'].reshape(1, -1), p['Wf1'], p['bf1'].reshape(1, -1),
      p['Wf2'], p['bf2'].reshape(1, -1), p['We'], p['be'].reshape(1, -1))


def kernel(x, edge_index, edge_attr, batch, global_feat, params):
    src = edge_index[0].astype(jnp.int32)
    dst = edge_index[1].astype(jnp.int32)
    batch2 = batch.astype(jnp.int32).reshape(N, 1)

    src2 = jnp.concatenate([src, src + N])

    h_split = x
    for l in range(4):
        layer0 = l == 0
        hin = 128 if layer0 else HID
        elin = _edge_lin(edge_attr, params['edge_W_%d' % l],
                         params['edge_b_%d' % l], 1 if layer0 else 2)
        aggr_split = _sc_aggr(layer0)(h_split, elin,
                                      src if layer0 else src2, dst)
        v, s1, s2 = _dense_a(h_split, aggr_split, params['W1_%d' % l],
                             params['b1_%d' % l], params['W2_%d' % l],
                             params['b2_%d' % l], hin, layer0)
        h_split = _dense_b(v, s1, s2, params['bn_g_%d' % l],
                           params['bn_b_%d' % l])

    edl, z, g = _head(h_split, batch2, global_feat, params)
    return (edl, z, g)
